# baseline (speedup 1.0000x reference)
import functools

import jax
import jax.numpy as jnp
from jax import lax
from jax.experimental import pallas as pl
from jax.experimental.pallas import tpu as pltpu
from jax.experimental.pallas import tpu_sc as plsc

D = 768
T = 77
B = 256

NC = 2
NS = 16
NW = NC * NS
BBLK = B // NW
ROWS_PER_W = T * BBLK
TC0 = 3
NBUF = 3
LANES = 16
GROUPS = D // LANES

_TCHUNKS = [(i * TC0, TC0) for i in range(T // TC0)]
if T % TC0:
    _TCHUNKS.append((T - T % TC0, T % TC0))


def _make_kernel():
    mesh = plsc.VectorSubcoreMesh(core_axis_name="c", subcore_axis_name="s")

    @functools.partial(
        pl.kernel,
        mesh=mesh,
        out_type=jax.ShapeDtypeStruct((T, B, D), jnp.float32),
        scratch_types=(
            [pltpu.VMEM((ROWS_PER_W,), jnp.int32),
             pltpu.VMEM((T, D), jnp.float32)]
            + [pltpu.VMEM((TC0 * BBLK, D), jnp.float32)] * NBUF
            + [pltpu.SemaphoreType.DMA] * (2 * NBUF)
        ),
    )
    def k(tokens_hbm, table_hbm, pos_hbm, out_hbm, idx_v, pos_v, *rest):
        bufs = rest[:NBUF]
        sems_g = rest[NBUF:2 * NBUF]
        sems_w = rest[2 * NBUF:]
        wid = lax.axis_index("s") * NC + lax.axis_index("c")
        b0 = wid * BBLK
        pltpu.sync_copy(tokens_hbm.at[pl.ds(wid * ROWS_PER_W, ROWS_PER_W)],
                        idx_v)
        pltpu.sync_copy(pos_hbm, pos_v)

        n = len(_TCHUNKS)
        gathers = {}
        writes = {}

        def issue_gather(ci):
            t0, tc = _TCHUNKS[ci]
            gathers[ci] = pltpu.async_copy(
                table_hbm.at[idx_v.at[pl.ds(t0 * BBLK, tc * BBLK)]],
                bufs[ci % NBUF].at[pl.ds(0, tc * BBLK)],
                sems_g[ci % NBUF],
            )

        issue_gather(0)
        for ci, (t0, tc) in enumerate(_TCHUNKS):
            buf = bufs[ci % NBUF]
            gathers[ci].wait()
            if ci >= 2:
                for w in writes[ci - 2]:
                    w.wait()
            if ci + 1 < n:
                issue_gather(ci + 1)

            def body(j, _):
                r0 = j * BBLK

                def gbody(g, _):
                    sl = pl.ds(g * LANES, LANES)
                    pv = pos_v[t0 + j, sl]
                    for jj in range(BBLK):
                        plsc.addupdate(buf.at[r0 + jj, sl], pv)
                    return 0

                lax.fori_loop(0, GROUPS, gbody, 0)
                return 0

            lax.fori_loop(0, tc, body, 0)
            writes[ci] = [
                pltpu.async_copy(
                    buf.at[pl.ds(j * BBLK, BBLK)],
                    out_hbm.at[t0 + j, pl.ds(b0, BBLK)],
                    sems_w[ci % NBUF],
                )
                for j in range(tc)
            ]
        for ci in (n - 2, n - 1):
            for w in writes[ci]:
                w.wait()

    return k


_grid_kernel = _make_kernel()


def kernel(tokens, token_embedding, position_embedding):
    tok = (tokens.astype(jnp.int32).T
           .reshape(T, NW, BBLK).transpose(1, 0, 2).reshape(-1))
    out = _grid_kernel(tok, token_embedding, position_embedding)
    return out.transpose(1, 0, 2)
